# Initial kernel scaffold; baseline (speedup 1.0000x reference)
import jax
import jax.numpy as jnp
from jax.experimental import pallas as pl


def kernel(x, edge_types, t, means, temps, mul_w, bias_w):
    raise NotImplementedError("write your pallas kernel here")


# speedup vs baseline: 6.9561x; 6.9561x over previous
import functools

import jax
import jax.numpy as jnp
from jax import lax
from jax.experimental import pallas as pl
from jax.experimental.pallas import tpu as pltpu
from jax.experimental.pallas import tpu_sc as plsc

E = 320000
K = 128
N_EDGE_TYPES = 1024


_NC = 2
_NS = 16
_NW = _NC * _NS
_CHUNK = E // _NW
_LANES = 16


def _sc_affine_body(x_hbm, et_hbm, mul_hbm, bias_hbm, out_hbm,
                    idx_v, x_v, mul_v, bias_v, out_v, sem):
    wid = lax.axis_index("s") * _NC + lax.axis_index("c")
    base = wid * _CHUNK
    pltpu.sync_copy(et_hbm.at[pl.ds(base, _CHUNK)], idx_v)
    pltpu.sync_copy(x_hbm.at[pl.ds(base, _CHUNK)], x_v)
    pltpu.async_copy(mul_hbm.at[idx_v], mul_v, sem).wait()
    pltpu.async_copy(bias_hbm.at[idx_v], bias_v, sem).wait()

    def body(i, carry):
        sl = pl.ds(pl.multiple_of(i * _LANES, _LANES), _LANES)
        out_v[sl] = mul_v[sl] * x_v[sl] + bias_v[sl]
        return carry

    lax.fori_loop(0, _CHUNK // _LANES, body, jnp.int32(0))
    pltpu.sync_copy(out_v, out_hbm.at[pl.ds(base, _CHUNK)])


_sc_affine = functools.partial(
    pl.kernel,
    mesh=plsc.VectorSubcoreMesh(core_axis_name="c", subcore_axis_name="s"),
    out_type=jax.ShapeDtypeStruct((E,), jnp.float32),
    scratch_types=[
        pltpu.VMEM((_CHUNK,), jnp.int32),
        pltpu.VMEM((_CHUNK,), jnp.float32),
        pltpu.VMEM((_CHUNK,), jnp.float32),
        pltpu.VMEM((_CHUNK,), jnp.float32),
        pltpu.VMEM((_CHUNK,), jnp.float32),
        pltpu.SemaphoreType.DMA,
    ],
)(_sc_affine_body)


_BE = 4000


def _rbf_body(xe_ref, m_ref, t_ref, o_ref):
    xe = xe_ref[...]
    m = m_ref[...]
    t = jnp.abs(t_ref[...])
    d = xe - m
    o_ref[...] = jnp.exp(d * d * (-t))


def _rbf_dense(xe2, mean0, temp0):
    return pl.pallas_call(
        _rbf_body,
        grid=(E // _BE,),
        in_specs=[
            pl.BlockSpec((_BE, 1), lambda i: (i, 0)),
            pl.BlockSpec((1, K), lambda i: (0, 0)),
            pl.BlockSpec((1, K), lambda i: (0, 0)),
        ],
        out_specs=pl.BlockSpec((_BE, K), lambda i: (i, 0)),
        out_shape=jax.ShapeDtypeStruct((E, K), jnp.float32),
    )(xe2, mean0, temp0)


def kernel(x, edge_types, t, means, temps, mul_w, bias_w):
    xe = _sc_affine(x, edge_types, mul_w.reshape(-1), bias_w.reshape(-1))
    mean0 = lax.slice(means, (0, 0), (1, K))
    temp0 = lax.slice(temps, (0, 0), (1, K))
    return _rbf_dense(xe.reshape(E, 1), mean0, temp0)


# speedup vs baseline: 8.6041x; 1.2369x over previous
import functools

import jax
import jax.numpy as jnp
from jax import lax
from jax.experimental import pallas as pl
from jax.experimental.pallas import tpu as pltpu
from jax.experimental.pallas import tpu_sc as plsc

E = 320000
K = 128
N_EDGE_TYPES = 1024


_NC = 2
_NS = 16
_NW = _NC * _NS
_CHUNK = E // _NW
_LANES = 16


def _sc_affine_body(x_hbm, et_hbm, mul_hbm, bias_hbm, out_hbm,
                    idx_v, x_v, mul_v, bias_v, out_v, sem):
    wid = lax.axis_index("s") * _NC + lax.axis_index("c")
    base = wid * _CHUNK
    pltpu.sync_copy(et_hbm.at[pl.ds(base, _CHUNK)], idx_v)
    pltpu.sync_copy(x_hbm.at[pl.ds(base, _CHUNK)], x_v)
    pltpu.async_copy(mul_hbm.at[idx_v], mul_v, sem).wait()
    pltpu.async_copy(bias_hbm.at[idx_v], bias_v, sem).wait()

    def body(i, carry):
        sl = pl.ds(pl.multiple_of(i * _LANES, _LANES), _LANES)
        out_v[sl] = mul_v[sl] * x_v[sl] + bias_v[sl]
        return carry

    lax.fori_loop(0, _CHUNK // _LANES, body, jnp.int32(0))
    pltpu.sync_copy(out_v, out_hbm.at[pl.ds(base, _CHUNK)])


_sc_affine = functools.partial(
    pl.kernel,
    mesh=plsc.VectorSubcoreMesh(core_axis_name="c", subcore_axis_name="s"),
    out_type=jax.ShapeDtypeStruct((E,), jnp.float32),
    scratch_types=[
        pltpu.VMEM((_CHUNK,), jnp.int32),
        pltpu.VMEM((_CHUNK,), jnp.float32),
        pltpu.VMEM((_CHUNK,), jnp.float32),
        pltpu.VMEM((_CHUNK,), jnp.float32),
        pltpu.VMEM((_CHUNK,), jnp.float32),
        pltpu.SemaphoreType.DMA,
    ],
)(_sc_affine_body)


_BE = 3200


def _rbf_body(xe_ref, m_ref, t_ref, o_ref):
    xe = xe_ref[0]
    m = m_ref[...]
    t = jnp.abs(t_ref[...])
    xet = jnp.transpose(xe)
    d = xet - m
    o_ref[...] = jnp.exp(d * d * (-t))


def _rbf_dense(xe2, mean0, temp0):
    return pl.pallas_call(
        _rbf_body,
        grid=(E // _BE,),
        in_specs=[
            pl.BlockSpec((1, 1, _BE), lambda i: (i, 0, 0)),
            pl.BlockSpec((1, K), lambda i: (0, 0)),
            pl.BlockSpec((1, K), lambda i: (0, 0)),
        ],
        out_specs=pl.BlockSpec((_BE, K), lambda i: (i, 0)),
        out_shape=jax.ShapeDtypeStruct((E, K), jnp.float32),
    )(xe2, mean0, temp0)


def kernel(x, edge_types, t, means, temps, mul_w, bias_w):
    xe = _sc_affine(x, edge_types, mul_w.reshape(-1), bias_w.reshape(-1))
    mean0 = lax.slice(means, (0, 0), (1, K))
    temp0 = lax.slice(temps, (0, 0), (1, K))
    return _rbf_dense(xe.reshape(E // _BE, 1, _BE), mean0, temp0)


# speedup vs baseline: 30.3603x; 3.5286x over previous
import functools

import jax
import jax.numpy as jnp
from jax import lax
from jax.experimental import pallas as pl
from jax.experimental.pallas import tpu as pltpu
from jax.experimental.pallas import tpu_sc as plsc

E = 320000
K = 128
N_EDGE_TYPES = 1024


_NC = 2
_NS = 16
_NW = _NC * _NS
_CHUNK = E // _NW
_LANES = 16


def _sc_affine_body(x_hbm, et_hbm, mul_hbm, bias_hbm, out_hbm,
                    idx_v, x_v, mul_v, bias_v, out_v, mul_t, bias_t, sem):
    wid = lax.axis_index("s") * _NC + lax.axis_index("c")
    base = wid * _CHUNK
    pltpu.sync_copy(et_hbm.at[pl.ds(base, _CHUNK)], idx_v)
    pltpu.sync_copy(x_hbm.at[pl.ds(base, _CHUNK)], x_v)
    @pl.when(lax.axis_index("s") == 0)
    def _stage_tables():
        pltpu.sync_copy(mul_hbm, mul_t)
        pltpu.sync_copy(bias_hbm, bias_t)

    plsc.subcore_barrier()
    pltpu.async_copy(mul_t.at[idx_v], mul_v, sem).wait()
    pltpu.async_copy(bias_t.at[idx_v], bias_v, sem).wait()

    def body(i, carry):
        sl = pl.ds(pl.multiple_of(i * _LANES, _LANES), _LANES)
        out_v[sl] = mul_v[sl] * x_v[sl] + bias_v[sl]
        return carry

    lax.fori_loop(0, _CHUNK // _LANES, body, jnp.int32(0))
    pltpu.sync_copy(out_v, out_hbm.at[pl.ds(base, _CHUNK)])


_sc_affine = functools.partial(
    pl.kernel,
    mesh=plsc.VectorSubcoreMesh(core_axis_name="c", subcore_axis_name="s"),
    out_type=jax.ShapeDtypeStruct((E,), jnp.float32),
    scratch_types=[
        pltpu.VMEM((_CHUNK,), jnp.int32),
        pltpu.VMEM((_CHUNK,), jnp.float32),
        pltpu.VMEM((_CHUNK,), jnp.float32),
        pltpu.VMEM((_CHUNK,), jnp.float32),
        pltpu.VMEM((_CHUNK,), jnp.float32),
        pltpu.VMEM_SHARED((N_EDGE_TYPES,), jnp.float32),
        pltpu.VMEM_SHARED((N_EDGE_TYPES,), jnp.float32),
        pltpu.SemaphoreType.DMA,
    ],
)(_sc_affine_body)


_BE = 3200


def _rbf_body(xe_ref, m_ref, t_ref, o_ref):
    xe = xe_ref[0]
    m = m_ref[...]
    t = jnp.abs(t_ref[...])
    xet = jnp.transpose(xe)
    d = xet - m
    o_ref[...] = jnp.exp(d * d * (-t))


def _rbf_dense(xe2, mean0, temp0):
    return pl.pallas_call(
        _rbf_body,
        grid=(E // _BE,),
        in_specs=[
            pl.BlockSpec((1, 1, _BE), lambda i: (i, 0, 0)),
            pl.BlockSpec((1, K), lambda i: (0, 0)),
            pl.BlockSpec((1, K), lambda i: (0, 0)),
        ],
        out_specs=pl.BlockSpec((_BE, K), lambda i: (i, 0)),
        out_shape=jax.ShapeDtypeStruct((E, K), jnp.float32),
    )(xe2, mean0, temp0)


def kernel(x, edge_types, t, means, temps, mul_w, bias_w):
    xe = _sc_affine(x, edge_types, mul_w.reshape(-1), bias_w.reshape(-1))
    mean0 = lax.slice(means, (0, 0), (1, K))
    temp0 = lax.slice(temps, (0, 0), (1, K))
    return _rbf_dense(xe.reshape(E // _BE, 1, _BE), mean0, temp0)


# speedup vs baseline: 35.1163x; 1.1566x over previous
import functools

import jax
import jax.numpy as jnp
from jax import lax
from jax.experimental import pallas as pl
from jax.experimental.pallas import tpu as pltpu
from jax.experimental.pallas import tpu_sc as plsc

E = 320000
K = 128
N_EDGE_TYPES = 1024


_NC = 2
_NS = 16
_NW = _NC * _NS
_CHUNK = E // _NW
_LANES = 16


def _sc_affine_body(x_hbm, et_hbm, mul_hbm, bias_hbm, out_hbm,
                    idx_v, x_v, mul_v, bias_v, out_v, mul_t, bias_t, sem):
    wid = lax.axis_index("s") * _NC + lax.axis_index("c")
    base = wid * _CHUNK
    pltpu.sync_copy(et_hbm.at[pl.ds(base, _CHUNK)], idx_v)
    pltpu.sync_copy(x_hbm.at[pl.ds(base, _CHUNK)], x_v)
    @pl.when(lax.axis_index("s") == 0)
    def _stage_tables():
        pltpu.sync_copy(mul_hbm, mul_t)
        pltpu.sync_copy(bias_hbm, bias_t)

    plsc.subcore_barrier()
    pltpu.async_copy(mul_t.at[idx_v], mul_v, sem).wait()
    pltpu.async_copy(bias_t.at[idx_v], bias_v, sem).wait()

    def body(i, carry):
        sl = pl.ds(pl.multiple_of(i * _LANES, _LANES), _LANES)
        out_v[sl] = mul_v[sl] * x_v[sl] + bias_v[sl]
        return carry

    lax.fori_loop(0, _CHUNK // _LANES, body, jnp.int32(0))
    pltpu.sync_copy(out_v, out_hbm.at[pl.ds(base, _CHUNK)])


_sc_affine = functools.partial(
    pl.kernel,
    mesh=plsc.VectorSubcoreMesh(core_axis_name="c", subcore_axis_name="s"),
    out_type=jax.ShapeDtypeStruct((E,), jnp.float32),
    scratch_types=[
        pltpu.VMEM((_CHUNK,), jnp.int32),
        pltpu.VMEM((_CHUNK,), jnp.float32),
        pltpu.VMEM((_CHUNK,), jnp.float32),
        pltpu.VMEM((_CHUNK,), jnp.float32),
        pltpu.VMEM((_CHUNK,), jnp.float32),
        pltpu.VMEM_SHARED((N_EDGE_TYPES,), jnp.float32),
        pltpu.VMEM_SHARED((N_EDGE_TYPES,), jnp.float32),
        pltpu.SemaphoreType.DMA,
    ],
)(_sc_affine_body)


_BE = 12800
_SUB = 512


def _rbf_body(xe_ref, m_ref, t_ref, o_ref):
    m = m_ref[...]
    nt = jnp.abs(t_ref[...]) * jnp.float32(-1.4426950408889634)
    xe = xe_ref[0]
    xet = jnp.transpose(xe)
    d = xet - m
    o_ref[...] = jnp.exp2(d * d * nt)


def _rbf_dense(xe2, mean0, temp0):
    return pl.pallas_call(
        _rbf_body,
        grid=(E // _BE,),
        in_specs=[
            pl.BlockSpec((1, 1, _BE), lambda i: (i, 0, 0)),
            pl.BlockSpec((1, K), lambda i: (0, 0)),
            pl.BlockSpec((1, K), lambda i: (0, 0)),
        ],
        out_specs=pl.BlockSpec((_BE, K), lambda i: (i, 0)),
        out_shape=jax.ShapeDtypeStruct((E, K), jnp.float32),
    )(xe2, mean0, temp0)


def kernel(x, edge_types, t, means, temps, mul_w, bias_w):
    xe = _sc_affine(x, edge_types, mul_w.reshape(-1), bias_w.reshape(-1))
    mean0 = lax.slice(means, (0, 0), (1, K))
    temp0 = lax.slice(temps, (0, 0), (1, K))
    return _rbf_dense(xe.reshape(E // _BE, 1, _BE), mean0, temp0)
